# baseline (speedup 1.0000x reference)
import jax
import jax.numpy as jnp
from jax import lax
from jax.experimental import pallas as pl
from jax.experimental.pallas import tpu as pltpu
from jax.experimental.pallas import tpu_sc as plsc

N = 10000
E = 320000
D = 128
MSG_EPS = 1e-7
BN_EPS = 1e-5

NW = 32
BW = 320
NEG = -3e38
CBIN = 4000
FB = 2048
OBUF = FB + CBIN + 96
EP = (E // FB + 1) * FB
C2 = 256


def _cumsum16(x):
    iota = lax.broadcasted_iota(jnp.int32, (16,), 0)
    for s in (1, 2, 4, 8):
        idx = jnp.maximum(iota - s, 0)
        shifted = lax.gather(
            x, idx[:, None],
            lax.GatherDimensionNumbers(offset_dims=(),
                                       collapsed_slice_dims=(0,),
                                       start_index_map=(0,)),
            slice_sizes=(1,),
            mode=lax.GatherScatterMode.PROMISE_IN_BOUNDS)
        x = x + jnp.where(iota >= s, shifted, 0)
    return x


def _bin_body(src_hbm, dst_hbm, sb, db, cnts, sbuf, dbuf, osrc, odst, cbuf):
    wid = lax.axis_index("s") * 2 + lax.axis_index("c")
    lo = wid * BW
    obase = wid * EP

    def chunk(c, carry):
        fill, cur = carry
        base = pl.multiple_of(c * CBIN, 8)
        pltpu.sync_copy(src_hbm.at[pl.ds(base, CBIN)], sbuf)
        pltpu.sync_copy(dst_hbm.at[pl.ds(base, CBIN)], dbuf)

        def scan(k, fill):
            v = dbuf[pl.ds(k * 16, 16)]
            s = sbuf[pl.ds(k * 16, 16)]
            a = v - lo
            c1 = 1 + lax.shift_right_arithmetic(a, 31)
            c2 = 1 + lax.shift_right_arithmetic(BW - 1 - a, 31)
            ones = c1 * c2
            csum = _cumsum16(ones)
            cbuf[pl.ds(0, 16)] = fill + csum - 1
            pos = cbuf[pl.ds(0, 16)]
            m = ones > 0
            plsc.store_scatter(odst, [pos], a, mask=m)
            plsc.store_scatter(osrc, [pos], s, mask=m)
            return fill + csum[15]

        fill = lax.fori_loop(0, CBIN // 16, scan, fill)

        @pl.when(fill >= FB)
        def _():
            off = pl.multiple_of(obase + cur, FB)
            pltpu.sync_copy(osrc.at[pl.ds(0, FB)], sb.at[pl.ds(off, FB)])
            pltpu.sync_copy(odst.at[pl.ds(0, FB)], db.at[pl.ds(off, FB)])
            for i in range((CBIN + 16) // 16):
                osrc[pl.ds(i * 16, 16)] = osrc[pl.ds(FB + i * 16, 16)]
                odst[pl.ds(i * 16, 16)] = odst[pl.ds(FB + i * 16, 16)]

        flushed = jnp.where(fill >= FB, FB, 0)
        return fill - flushed, cur + flushed

    fill, cur = lax.fori_loop(0, E // CBIN, chunk, (jnp.int32(0), jnp.int32(0)))

    off = pl.multiple_of(obase + cur, FB)
    pltpu.sync_copy(osrc.at[pl.ds(0, FB)], sb.at[pl.ds(off, FB)])
    pltpu.sync_copy(odst.at[pl.ds(0, FB)], db.at[pl.ds(off, FB)])

    total = cur + fill
    cbuf[pl.ds(0, 16)] = jnp.full((16,), 0, jnp.int32) + total
    coff = pl.multiple_of(wid * 8, 8)
    pltpu.sync_copy(cbuf.at[pl.ds(0, 8)], cnts.at[pl.ds(coff, 8)])


def _bin_edges(src, dst):
    mesh = plsc.VectorSubcoreMesh(core_axis_name="c", subcore_axis_name="s")
    f = pl.kernel(
        _bin_body,
        out_type=(jax.ShapeDtypeStruct((NW * EP,), jnp.int32),
                  jax.ShapeDtypeStruct((NW * EP,), jnp.int32),
                  jax.ShapeDtypeStruct((NW * 8,), jnp.int32)),
        mesh=mesh,
        compiler_params=pltpu.CompilerParams(needs_layout_passes=False),
        scratch_types=[
            pltpu.VMEM((CBIN,), jnp.int32),
            pltpu.VMEM((CBIN,), jnp.int32),
            pltpu.VMEM((OBUF,), jnp.int32),
            pltpu.VMEM((OBUF,), jnp.int32),
            pltpu.VMEM((16,), jnp.int32),
        ],
    )
    return f(src, dst)


SUP = 2048
NCH_SUP = SUP // C2


def _agg_body(h_hbm, sb, db, cnts, out_hbm, sidx, dbuf, rows0, rows1,
              acc, cbuf, sem0, sem1):
    wid = lax.axis_index("s") * 2 + lax.axis_index("c")
    lo = wid * BW
    ebase = wid * EP
    rows = (rows0, rows1)
    sems = (sem0, sem1)

    coff = pl.multiple_of(wid * 8, 8)
    pltpu.sync_copy(cnts.at[pl.ds(coff, 8)], cbuf.at[pl.ds(0, 8)])
    cnt = cbuf[pl.ds(0, 16)][0]

    def init(i, _):
        acc[pl.ds(i * 16, 16)] = jnp.full((16,), NEG, jnp.float32)
        return 0
    lax.fori_loop(0, (BW + 1) * D // 16, init, 0)

    nch = (cnt + C2 - 1) // C2
    nsup = (cnt + SUP - 1) // SUP

    def gather_start(jc, b):
        pltpu.async_copy(
            h_hbm.at[sidx.at[pl.ds(jc * C2, C2)]], rows[b], sems[b])

    def gather_wait(b):
        pltpu.make_async_copy(
            h_hbm.at[sidx.at[pl.ds(0, C2)]], rows[b], sems[b]).wait()

    def process(ci, jc, b):
        rbuf = rows[b]
        base16 = jc * (C2 // 16)

        def extracts(kb):
            dvec = dbuf[pl.ds((base16 + kb) * 16, 16)] * D
            return tuple(dvec[i] for i in range(16))

        def edge_block(kb, carry):
            for i in range(16):
                doff = carry[i]
                e = kb * 16 + i
                for j in range(D // 16):
                    a = acc[pl.ds(doff + j * 16, 16)]
                    r = rbuf[e, pl.ds(j * 16, 16)]
                    acc[pl.ds(doff + j * 16, 16)] = jnp.maximum(a, r)
            return extracts(kb + 1)

        lax.fori_loop(0, C2 // 16, edge_block, extracts(0))

    def superchunk(sc, _):
        sbase = sc * SUP
        eoff = pl.multiple_of(ebase + sbase, SUP)
        pltpu.sync_copy(sb.at[pl.ds(eoff, SUP)], sidx.at[pl.ds(0, SUP)])
        pltpu.sync_copy(db.at[pl.ds(eoff, SUP)], dbuf.at[pl.ds(0, SUP)])

        rem = cnt - sbase

        def sani(k, _):
            idx16 = k * 16 + lax.broadcasted_iota(jnp.int32, (16,), 0)
            valid = idx16 < rem
            sidx[pl.ds(k * 16, 16)] = jnp.where(valid, sidx[pl.ds(k * 16, 16)], 0)
            dbuf[pl.ds(k * 16, 16)] = jnp.where(valid, dbuf[pl.ds(k * 16, 16)], BW)
            return 0
        lax.fori_loop(0, SUP // 16, sani, 0)

        cbase = sc * NCH_SUP

        @pl.when(cbase < nch)
        def _():
            gather_start(0, 0)

        for jc in range(NCH_SUP):
            b = jc % 2
            ci = cbase + jc

            @pl.when(ci + 1 < nch)
            def _(jc=jc, b=b):
                if jc + 1 < NCH_SUP:
                    gather_start(jc + 1, 1 - b)

            @pl.when(ci < nch)
            def _(jc=jc, b=b, ci=ci):
                gather_wait(b)
                process(ci, jc, b)

        return 0

    lax.fori_loop(0, nsup, superchunk, 0)

    @pl.when(wid < NW - 1)
    def _():
        pltpu.sync_copy(acc.at[pl.ds(0, BW * D)],
                        out_hbm.at[pl.ds(pl.multiple_of(lo * D, 8), BW * D)])

    @pl.when(wid == NW - 1)
    def _():
        nlast = N - BW * (NW - 1)
        pltpu.sync_copy(acc.at[pl.ds(0, nlast * D)],
                        out_hbm.at[pl.ds(pl.multiple_of(lo * D, 8), nlast * D)])


def _aggregate(h, sb, db, cnts):
    mesh = plsc.VectorSubcoreMesh(core_axis_name="c", subcore_axis_name="s")
    f = pl.kernel(
        _agg_body,
        out_type=jax.ShapeDtypeStruct((N * D,), jnp.float32),
        mesh=mesh,
        scratch_types=[
            pltpu.VMEM((SUP,), jnp.int32),
            pltpu.VMEM((SUP + 16,), jnp.int32),
            pltpu.VMEM((C2, D), jnp.float32),
            pltpu.VMEM((C2, D), jnp.float32),
            pltpu.VMEM(((BW + 1) * D,), jnp.float32),
            pltpu.VMEM((16,), jnp.int32),
            pltpu.SemaphoreType.DMA,
            pltpu.SemaphoreType.DMA,
        ],
    )
    return f(h, sb, db, cnts).reshape(N, D)


def _dense_body(t_ref, mraw_ref, res_ref, w_ref, b_ref, g_ref, beta_ref,
                h_ref, tn_ref):
    mraw = mraw_ref[...]
    m = jnp.where(mraw <= NEG * 0.5, 0.0, jnp.maximum(mraw, 0.0) + MSG_EPS)
    hin = t_ref[...] + m
    h = jnp.dot(hin, w_ref[...], preferred_element_type=jnp.float32)
    h = h + b_ref[...] + res_ref[...]
    h_ref[...] = h
    mu = jnp.mean(h, axis=0, keepdims=True)
    var = jnp.mean((h - mu) * (h - mu), axis=0, keepdims=True)
    tn = (h - mu) * lax.rsqrt(var + BN_EPS) * g_ref[...] + beta_ref[...]
    tn_ref[...] = jnp.maximum(tn, 0.0)


def _dense(t, mraw, res, W, b, g, beta):
    f = pl.pallas_call(
        _dense_body,
        out_shape=(jax.ShapeDtypeStruct((N, D), jnp.float32),
                   jax.ShapeDtypeStruct((N, D), jnp.float32)),
    )
    return f(t, mraw, res, W, b.reshape(1, D), g.reshape(1, D),
             beta.reshape(1, D))


def kernel(x, edge_index, W0, b0, W1, b1, W2, b2, g0, beta0, g1, beta1,
           g2, beta2):
    src = edge_index[0]
    dst = edge_index[1]
    sb, db, cnts = _bin_edges(src, dst)

    zeros = jnp.zeros((N, D), jnp.float32)
    m0 = _aggregate(x, sb, db, cnts)
    h_a, t_a = _dense(x, m0, zeros, W0, b0, g0, beta0)
    m1 = _aggregate(t_a, sb, db, cnts)
    h_b, t_b = _dense(t_a, m1, h_a, W1, b1, g1, beta1)
    m2 = _aggregate(t_b, sb, db, cnts)
    _, t_c = _dense(t_b, m2, h_b, W2, b2, g2, beta2)
    return t_c


# speedup vs baseline: 1.5196x; 1.5196x over previous
import jax
import jax.numpy as jnp
from jax import lax
from jax.experimental import pallas as pl
from jax.experimental.pallas import tpu as pltpu
from jax.experimental.pallas import tpu_sc as plsc

N = 10000
E = 320000
D = 128
MSG_EPS = 1e-7
BN_EPS = 1e-5

NW = 32
BW = 320
NEG = -3e38
CBIN = 4000
FB = 2048
OBUF = FB + CBIN + 96
EP = (E // FB + 1) * FB
C2 = 128


def _cumsum16(x):
    iota = lax.broadcasted_iota(jnp.int32, (16,), 0)
    for s in (1, 2, 4, 8):
        idx = jnp.maximum(iota - s, 0)
        shifted = lax.gather(
            x, idx[:, None],
            lax.GatherDimensionNumbers(offset_dims=(),
                                       collapsed_slice_dims=(0,),
                                       start_index_map=(0,)),
            slice_sizes=(1,),
            mode=lax.GatherScatterMode.PROMISE_IN_BOUNDS)
        x = x + jnp.where(iota >= s, shifted, 0)
    return x


def _bin_body(src_hbm, dst_hbm, sb, db, cnts, sbuf, dbuf, osrc, odst, cbuf):
    wid = lax.axis_index("s") * 2 + lax.axis_index("c")
    lo = wid * BW
    obase = wid * EP

    def chunk(c, carry):
        fill, cur = carry
        base = pl.multiple_of(c * CBIN, 8)
        pltpu.sync_copy(src_hbm.at[pl.ds(base, CBIN)], sbuf)
        pltpu.sync_copy(dst_hbm.at[pl.ds(base, CBIN)], dbuf)

        def scan(k, fill):
            v = dbuf[pl.ds(k * 16, 16)]
            s = sbuf[pl.ds(k * 16, 16)]
            a = v - lo
            c1 = 1 + lax.shift_right_arithmetic(a, 31)
            c2 = 1 + lax.shift_right_arithmetic(BW - 1 - a, 31)
            ones = c1 * c2
            csum = _cumsum16(ones)
            cbuf[pl.ds(0, 16)] = fill + csum - 1
            pos = cbuf[pl.ds(0, 16)]
            m = ones > 0
            plsc.store_scatter(odst, [pos], a, mask=m)
            plsc.store_scatter(osrc, [pos], s, mask=m)
            return fill + csum[15]

        fill = lax.fori_loop(0, CBIN // 16, scan, fill)

        @pl.when(fill >= FB)
        def _():
            off = pl.multiple_of(obase + cur, FB)
            pltpu.sync_copy(osrc.at[pl.ds(0, FB)], sb.at[pl.ds(off, FB)])
            pltpu.sync_copy(odst.at[pl.ds(0, FB)], db.at[pl.ds(off, FB)])
            for i in range((CBIN + 16) // 16):
                osrc[pl.ds(i * 16, 16)] = osrc[pl.ds(FB + i * 16, 16)]
                odst[pl.ds(i * 16, 16)] = odst[pl.ds(FB + i * 16, 16)]

        flushed = jnp.where(fill >= FB, FB, 0)
        return fill - flushed, cur + flushed

    fill, cur = lax.fori_loop(0, E // CBIN, chunk, (jnp.int32(0), jnp.int32(0)))

    off = pl.multiple_of(obase + cur, FB)
    pltpu.sync_copy(osrc.at[pl.ds(0, FB)], sb.at[pl.ds(off, FB)])
    pltpu.sync_copy(odst.at[pl.ds(0, FB)], db.at[pl.ds(off, FB)])

    total = cur + fill
    cbuf[pl.ds(0, 16)] = jnp.full((16,), 0, jnp.int32) + total
    coff = pl.multiple_of(wid * 8, 8)
    pltpu.sync_copy(cbuf.at[pl.ds(0, 8)], cnts.at[pl.ds(coff, 8)])


def _bin_edges(src, dst):
    mesh = plsc.VectorSubcoreMesh(core_axis_name="c", subcore_axis_name="s")
    f = pl.kernel(
        _bin_body,
        out_type=(jax.ShapeDtypeStruct((NW * EP,), jnp.int32),
                  jax.ShapeDtypeStruct((NW * EP,), jnp.int32),
                  jax.ShapeDtypeStruct((NW * 8,), jnp.int32)),
        mesh=mesh,
        compiler_params=pltpu.CompilerParams(needs_layout_passes=False),
        scratch_types=[
            pltpu.VMEM((CBIN,), jnp.int32),
            pltpu.VMEM((CBIN,), jnp.int32),
            pltpu.VMEM((OBUF,), jnp.int32),
            pltpu.VMEM((OBUF,), jnp.int32),
            pltpu.VMEM((16,), jnp.int32),
        ],
    )
    return f(src, dst)


SUP = 2048
NCH_SUP = SUP // C2


def _agg_body(h_hbm, sb, db, cnts, out_hbm, sidx, dbuf, rows0, rows1,
              acc, acc1, cbuf, sem0, sem1):
    wid = lax.axis_index("s") * 2 + lax.axis_index("c")
    lo = wid * BW
    ebase = wid * EP
    rows = (rows0, rows1)
    sems = (sem0, sem1)

    coff = pl.multiple_of(wid * 8, 8)
    pltpu.sync_copy(cnts.at[pl.ds(coff, 8)], cbuf.at[pl.ds(0, 8)])
    cnt = cbuf[pl.ds(0, 16)][0]

    def init(i, _):
        acc[pl.ds(i * 16, 16)] = jnp.full((16,), NEG, jnp.float32)
        acc1[pl.ds(i * 16, 16)] = jnp.full((16,), NEG, jnp.float32)
        return 0
    lax.fori_loop(0, (BW + 1) * D // 16, init, 0)

    nch = (cnt + C2 - 1) // C2
    nsup = (cnt + SUP - 1) // SUP

    def gather_start(jc, b):
        pltpu.async_copy(
            h_hbm.at[sidx.at[pl.ds(jc * C2, C2)]], rows[b], sems[b])

    def gather_wait(b):
        pltpu.make_async_copy(
            h_hbm.at[sidx.at[pl.ds(0, C2)]], rows[b], sems[b]).wait()

    def process(ci, jc, b):
        rbuf = rows[b]
        base16 = jc * (C2 // 16)

        def extracts(kb):
            dvec = dbuf[pl.ds((base16 + kb) * 16, 16)] * D
            return tuple(dvec[i] for i in range(16))

        def edge_block(kb, carry):
            for i in range(16):
                doff = carry[i]
                e = kb * 16 + i
                ac = acc if i % 2 == 0 else acc1
                rv = [rbuf[e, pl.ds(j * 16, 16)] for j in range(D // 16)]
                av = [ac[pl.ds(doff + j * 16, 16)] for j in range(D // 16)]
                for j in range(D // 16):
                    ac[pl.ds(doff + j * 16, 16)] = jnp.maximum(av[j], rv[j])
            return extracts(kb + 1)

        lax.fori_loop(0, C2 // 16, edge_block, extracts(0))

    def superchunk(sc, _):
        sbase = sc * SUP
        eoff = pl.multiple_of(ebase + sbase, SUP)
        pltpu.sync_copy(sb.at[pl.ds(eoff, SUP)], sidx.at[pl.ds(0, SUP)])
        pltpu.sync_copy(db.at[pl.ds(eoff, SUP)], dbuf.at[pl.ds(0, SUP)])

        rem = cnt - sbase

        def sani(k, _):
            idx16 = k * 16 + lax.broadcasted_iota(jnp.int32, (16,), 0)
            valid = idx16 < rem
            sidx[pl.ds(k * 16, 16)] = jnp.where(valid, sidx[pl.ds(k * 16, 16)], 0)
            dbuf[pl.ds(k * 16, 16)] = jnp.where(valid, dbuf[pl.ds(k * 16, 16)], BW)
            return 0
        lax.fori_loop(0, SUP // 16, sani, 0)

        cbase = sc * NCH_SUP

        @pl.when(cbase < nch)
        def _():
            gather_start(0, 0)

        for jc in range(NCH_SUP):
            b = jc % 2
            ci = cbase + jc

            @pl.when(ci + 1 < nch)
            def _(jc=jc, b=b):
                if jc + 1 < NCH_SUP:
                    gather_start(jc + 1, 1 - b)

            @pl.when(ci < nch)
            def _(jc=jc, b=b, ci=ci):
                gather_wait(b)
                process(ci, jc, b)

        return 0

    lax.fori_loop(0, nsup, superchunk, 0)

    def merge(i, _):
        a = acc[pl.ds(i * 16, 16)]
        b = acc1[pl.ds(i * 16, 16)]
        acc[pl.ds(i * 16, 16)] = jnp.maximum(a, b)
        return 0
    lax.fori_loop(0, (BW + 1) * D // 16, merge, 0)

    @pl.when(wid < NW - 1)
    def _():
        pltpu.sync_copy(acc.at[pl.ds(0, BW * D)],
                        out_hbm.at[pl.ds(pl.multiple_of(lo * D, 8), BW * D)])

    @pl.when(wid == NW - 1)
    def _():
        nlast = N - BW * (NW - 1)
        pltpu.sync_copy(acc.at[pl.ds(0, nlast * D)],
                        out_hbm.at[pl.ds(pl.multiple_of(lo * D, 8), nlast * D)])


def _aggregate(h, sb, db, cnts):
    mesh = plsc.VectorSubcoreMesh(core_axis_name="c", subcore_axis_name="s")
    f = pl.kernel(
        _agg_body,
        out_type=jax.ShapeDtypeStruct((N * D,), jnp.float32),
        mesh=mesh,
        scratch_types=[
            pltpu.VMEM((SUP,), jnp.int32),
            pltpu.VMEM((SUP + 16,), jnp.int32),
            pltpu.VMEM((C2, D), jnp.float32),
            pltpu.VMEM((C2, D), jnp.float32),
            pltpu.VMEM(((BW + 1) * D,), jnp.float32),
            pltpu.VMEM(((BW + 1) * D,), jnp.float32),
            pltpu.VMEM((16,), jnp.int32),
            pltpu.SemaphoreType.DMA,
            pltpu.SemaphoreType.DMA,
        ],
    )
    return f(h, sb, db, cnts).reshape(N, D)


def _dense_body(t_ref, mraw_ref, res_ref, w_ref, b_ref, g_ref, beta_ref,
                h_ref, tn_ref):
    mraw = mraw_ref[...]
    m = jnp.where(mraw <= NEG * 0.5, 0.0, jnp.maximum(mraw, 0.0) + MSG_EPS)
    hin = t_ref[...] + m
    h = jnp.dot(hin, w_ref[...], preferred_element_type=jnp.float32)
    h = h + b_ref[...] + res_ref[...]
    h_ref[...] = h
    mu = jnp.mean(h, axis=0, keepdims=True)
    var = jnp.mean((h - mu) * (h - mu), axis=0, keepdims=True)
    tn = (h - mu) * lax.rsqrt(var + BN_EPS) * g_ref[...] + beta_ref[...]
    tn_ref[...] = jnp.maximum(tn, 0.0)


def _dense(t, mraw, res, W, b, g, beta):
    f = pl.pallas_call(
        _dense_body,
        out_shape=(jax.ShapeDtypeStruct((N, D), jnp.float32),
                   jax.ShapeDtypeStruct((N, D), jnp.float32)),
    )
    return f(t, mraw, res, W, b.reshape(1, D), g.reshape(1, D),
             beta.reshape(1, D))


def kernel(x, edge_index, W0, b0, W1, b1, W2, b2, g0, beta0, g1, beta1,
           g2, beta2):
    src = edge_index[0]
    dst = edge_index[1]
    sb, db, cnts = _bin_edges(src, dst)

    zeros = jnp.zeros((N, D), jnp.float32)
    m0 = _aggregate(x, sb, db, cnts)
    h_a, t_a = _dense(x, m0, zeros, W0, b0, g0, beta0)
    m1 = _aggregate(t_a, sb, db, cnts)
    h_b, t_b = _dense(t_a, m1, h_a, W1, b1, g1, beta1)
    m2 = _aggregate(t_b, sb, db, cnts)
    _, t_c = _dense(t_b, m2, h_b, W2, b2, g2, beta2)
    return t_c
